# baseline (speedup 1.0000x reference)
import functools

import jax
import jax.numpy as jnp
from jax import lax
from jax.experimental import pallas as pl
from jax.experimental.pallas import tpu as pltpu
from jax.experimental.pallas import tpu_sc as plsc

BATCH = 16384
SEQ = 201
DIM = 64
SEQP = 208
PAIRS = SEQP // 2
BFLAT = BATCH * SEQP
NC, NS = 2, 16
NW = NC * NS
ROWS_PER_W = BFLAT // NW
CHUNK = 512
IW = 128
IPC = CHUNK // IW
CHUNKS_PER_W = ROWS_PER_W // CHUNK
NB = 2
OUTER = CHUNKS_PER_W // NB
BB = 256
NT = SEQP // 16


def _make_gather():
    mesh = plsc.VectorSubcoreMesh(core_axis_name="c", subcore_axis_name="s")

    @functools.partial(
        pl.kernel,
        mesh=mesh,
        out_type=jax.ShapeDtypeStruct((BFLAT, DIM), jnp.float32),
        scratch_types=[
            [pltpu.VMEM((IPC, IW), jnp.int32) for _ in range(NB)],
            [pltpu.VMEM((CHUNK, DIM), jnp.float32) for _ in range(NB)],
            [pltpu.SemaphoreType.DMA for _ in range(NB)],
            [pltpu.SemaphoreType.DMA for _ in range(NB)],
        ],
        compiler_params=pltpu.CompilerParams(use_tc_tiling_on_sc=False),
    )
    def gather_kernel(idx_hbm, table_hbm, out_hbm, idx_v, rows_v, gsem, ssem):
        wid = lax.axis_index("s") * NC + lax.axis_index("c")
        base = wid * CHUNKS_PER_W

        def fire_gathers(b, c):
            pltpu.sync_copy(idx_hbm.at[pl.ds((base + c) * IPC, IPC)], idx_v[b])
            for j in range(IPC):
                pltpu.async_copy(
                    table_hbm.at[idx_v[b].at[j]],
                    rows_v[b].at[pl.ds(j * IW, IW)],
                    gsem[b],
                )

        def wait_gathers(b):
            for j in range(IPC):
                pltpu.make_async_copy(
                    table_hbm.at[idx_v[b].at[j]],
                    rows_v[b].at[pl.ds(j * IW, IW)],
                    gsem[b],
                ).wait()

        def fire_store(b, c):
            pltpu.async_copy(
                rows_v[b], out_hbm.at[pl.ds((base + c) * CHUNK, CHUNK)], ssem[b]
            )

        def wait_store(b, c):
            pltpu.make_async_copy(
                rows_v[b], out_hbm.at[pl.ds((base + c) * CHUNK, CHUNK)], ssem[b]
            ).wait()

        for b in range(NB):
            fire_gathers(b, b)

        def body(g, carry):
            c0 = g * NB
            for b in range(NB):
                wait_gathers(b)
                fire_store(b, c0 + b)

            @pl.when(g < OUTER - 1)
            def _prefetch():
                for b in range(NB):
                    wait_store(b, c0 + b)
                    fire_gathers(b, c0 + NB + b)

            return carry

        lax.fori_loop(0, OUTER, body, 0)
        for b in range(NB):
            wait_store(b, (OUTER - 1) * NB + b)

    return gather_kernel


def _tr_kernel(x_ref, o_ref):
    x = x_ref[:, 0]
    y = jnp.transpose(x, (1, 2, 0))
    o_ref[...] = y.reshape(16, DIM, BB)


_transpose = pl.pallas_call(
    _tr_kernel,
    grid=(BATCH // BB, NT),
    in_specs=[pl.BlockSpec((BB, 1, 8, 128), lambda i, j: (i, j, 0, 0))],
    out_specs=pl.BlockSpec((16, DIM, BB), lambda i, j: (j, 0, i)),
    out_shape=jax.ShapeDtypeStruct((SEQ, DIM, BATCH), jnp.float32),
)

_gather = _make_gather()


@jax.jit
def kernel(w, table):
    idxp = jnp.pad(w.astype(jnp.int32), ((0, 0), (0, SEQP - SEQ)), mode="edge")
    oc = _gather(idxp.reshape(BFLAT // IW, IW), table)
    ot = _transpose(oc.reshape(BATCH, NT, 8, 128))
    return jnp.transpose(ot, (2, 0, 1))


# speedup vs baseline: 3.7153x; 3.7153x over previous
import functools

import jax
import jax.numpy as jnp
from jax import lax
from jax.experimental import pallas as pl
from jax.experimental.pallas import tpu as pltpu
from jax.experimental.pallas import tpu_sc as plsc

BATCH = 16384
SEQ = 201
DIM = 64
SEQP = 208
PAIRS = SEQP // 2
BFLAT = BATCH * SEQP
NC, NS = 2, 16
NW = NC * NS
ROWS_PER_W = BFLAT // NW
CHUNK = 512
IW = 128
IPC = CHUNK // IW
CHUNKS_PER_W = ROWS_PER_W // CHUNK
NB = 2
OUTER = CHUNKS_PER_W // NB
BB = 128
NT = SEQP // 16


def _make_gather():
    mesh = plsc.VectorSubcoreMesh(core_axis_name="c", subcore_axis_name="s")

    @functools.partial(
        pl.kernel,
        mesh=mesh,
        out_type=jax.ShapeDtypeStruct((BFLAT, DIM), jnp.float32),
        scratch_types=[
            [pltpu.VMEM((IPC, IW), jnp.int32) for _ in range(NB)],
            [pltpu.VMEM((CHUNK, DIM), jnp.float32) for _ in range(NB)],
            [pltpu.SemaphoreType.DMA for _ in range(NB)],
            [pltpu.SemaphoreType.DMA for _ in range(NB)],
        ],
        compiler_params=pltpu.CompilerParams(use_tc_tiling_on_sc=False),
    )
    def gather_kernel(idx_hbm, table_hbm, out_hbm, idx_v, rows_v, gsem, ssem):
        wid = lax.axis_index("s") * NC + lax.axis_index("c")
        base = wid * CHUNKS_PER_W

        def fire_gathers(b, c):
            pltpu.sync_copy(idx_hbm.at[pl.ds((base + c) * IPC, IPC)], idx_v[b])
            for j in range(IPC):
                pltpu.async_copy(
                    table_hbm.at[idx_v[b].at[j]],
                    rows_v[b].at[pl.ds(j * IW, IW)],
                    gsem[b],
                )

        def wait_gathers(b):
            for j in range(IPC):
                pltpu.make_async_copy(
                    table_hbm.at[idx_v[b].at[j]],
                    rows_v[b].at[pl.ds(j * IW, IW)],
                    gsem[b],
                ).wait()

        def fire_store(b, c):
            pltpu.async_copy(
                rows_v[b], out_hbm.at[pl.ds((base + c) * CHUNK, CHUNK)], ssem[b]
            )

        def wait_store(b, c):
            pltpu.make_async_copy(
                rows_v[b], out_hbm.at[pl.ds((base + c) * CHUNK, CHUNK)], ssem[b]
            ).wait()

        for b in range(NB):
            fire_gathers(b, b)

        def body(g, carry):
            c0 = g * NB
            for b in range(NB):
                wait_gathers(b)
                fire_store(b, c0 + b)

            @pl.when(g < OUTER - 1)
            def _prefetch():
                for b in range(NB):
                    wait_store(b, c0 + b)
                    fire_gathers(b, c0 + NB + b)

            return carry

        lax.fori_loop(0, OUTER, body, 0)
        for b in range(NB):
            wait_store(b, (OUTER - 1) * NB + b)

    return gather_kernel


def _tr_kernel(x_ref, o_ref):
    eye = jnp.eye(BB, dtype=jnp.float32)
    for u in range(8):
        xu = x_ref[:, 0, u, :]
        tu = jax.lax.dot_general(
            xu, eye, (((0,), (0,)), ((), ())),
            preferred_element_type=jnp.float32,
        )
        o_ref[2 * u : 2 * u + 2] = tu.reshape(2, DIM, BB)


_transpose = pl.pallas_call(
    _tr_kernel,
    grid=(BATCH // BB, NT),
    in_specs=[pl.BlockSpec((BB, 1, 8, 128), lambda i, j: (i, j, 0, 0))],
    out_specs=pl.BlockSpec((16, DIM, BB), lambda i, j: (j, 0, i)),
    out_shape=jax.ShapeDtypeStruct((SEQ, DIM, BATCH), jnp.float32),
)

_gather = _make_gather()


@jax.jit
def kernel(w, table):
    idxp = jnp.pad(w.astype(jnp.int32), ((0, 0), (0, SEQP - SEQ)), mode="edge")
    oc = _gather(idxp.reshape(BFLAT // IW, IW), table)
    ot = _transpose(oc.reshape(BATCH, NT, 8, 128))
    return jnp.transpose(ot, (2, 0, 1))
